# baseline (speedup 1.0000x reference)
import functools

import jax
import jax.numpy as jnp
from jax import lax
from jax.experimental import pallas as pl
from jax.experimental.pallas import tpu as pltpu
from jax.experimental.pallas import tpu_sc as plsc

B, C, N_IN = 4, 64, 8192
K, P = 4, 4
N_OUT = 32768
ROWS = B * C
NW = 32
ROWS_PER_W = ROWS // NW
NKROWS = N_IN * K
NPL = 2 * P
L = 16
Q = 2048
NQ = N_IN // Q


def _unpool_body(x_hbm, idx_hbm, planes_hbm, out_hbm,
                 acc, xr, irq, gpl, tables_sp, semA, semB):
    nc = 2
    wid = lax.axis_index("s") * nc + lax.axis_index("c")
    lane = jnp.arange(L, dtype=jnp.int32)
    zero = jnp.zeros((L,), jnp.float32)
    sems = [semA, semB]

    @pl.when(lax.axis_index("s") == 0)
    def _():
        pltpu.sync_copy(planes_hbm, tables_sp)

    plsc.subcore_barrier()

    def compute_quarter(q, buf):
        def inner(t, _):
            n0 = t * L
            xv = xr[pl.ds(q * Q + n0, L)]
            for p in range(P):
                smv = gpl[buf, p, pl.ds(n0, L)]
                iwv = plsc.bitcast(gpl[buf, P + p, pl.ds(n0, L)], jnp.float32)
                plsc.addupdate_scatter(acc, [smv], xv * iwv)
            return 0

        lax.fori_loop(0, Q // L, inner, 0)

    def row_body(i, _):
        r = wid * ROWS_PER_W + i
        bb = r // C
        cc = r % C
        pltpu.sync_copy(x_hbm.at[bb, cc], xr)

        copies = [None, None]
        for q in range(NQ):
            buf = q % 2
            pltpu.sync_copy(idx_hbm.at[bb, cc, pl.ds(q * Q, Q)], irq.at[buf])

            def gix_body(t, _):
                n0 = t * L
                kv = irq[buf, pl.ds(n0, L)]
                irq[buf, pl.ds(n0, L)] = (lane + (q * Q + n0)) * K + kv
                return 0

            lax.fori_loop(0, Q // L, gix_body, 0)
            cps = []
            for w in range(NPL):
                cp = pltpu.make_async_copy(
                    tables_sp.at[w].at[irq.at[buf]], gpl.at[buf, w], sems[buf])
                cp.start()
                cps.append(cp)
            copies[buf] = cps
            if q == 0:
                def zbody(j, _):
                    base = j * (L * 8)
                    for u in range(8):
                        acc[pl.ds(base + u * L, L)] = zero
                    return 0

                lax.fori_loop(0, N_OUT // (L * 8), zbody, 0)
            else:
                for cp in copies[1 - buf]:
                    cp.wait()
                compute_quarter(q - 1, 1 - buf)
        for cp in copies[(NQ - 1) % 2]:
            cp.wait()
        compute_quarter(NQ - 1, (NQ - 1) % 2)
        pltpu.sync_copy(acc, out_hbm.at[bb, cc])
        return 0

    lax.fori_loop(0, ROWS_PER_W, row_body, 0)


@jax.jit
def _unpool(x, idx, planes):
    mesh = plsc.VectorSubcoreMesh(core_axis_name="c", subcore_axis_name="s")
    f = functools.partial(
        pl.kernel,
        mesh=mesh,
        compiler_params=pltpu.CompilerParams(
            needs_layout_passes=False, use_tc_tiling_on_sc=False),
        out_type=jax.ShapeDtypeStruct((B, C, N_OUT), jnp.float32),
        scratch_types=[
            pltpu.VMEM((N_OUT,), jnp.float32),
            pltpu.VMEM((N_IN,), jnp.float32),
            pltpu.VMEM((2, Q), jnp.int32),
            pltpu.VMEM((2, NPL, Q), jnp.int32),
            pltpu.VMEM_SHARED((NPL, NKROWS), jnp.int32),
            pltpu.SemaphoreType.DMA,
            pltpu.SemaphoreType.DMA,
        ],
    )(_unpool_body)
    return f(x, idx, planes)


def kernel(x, idx_mask, sample_map, interp_weights):
    idxf = idx_mask.astype(jnp.int32)
    smT = sample_map.reshape(NKROWS, P).astype(jnp.int32).T
    iwT = lax.bitcast_convert_type(
        interp_weights.reshape(NKROWS, P), jnp.int32).T
    planes = jnp.concatenate([smT, iwT], axis=0)
    return _unpool(x, idxf, planes)


# speedup vs baseline: 1.2632x; 1.2632x over previous
import functools

import jax
import jax.numpy as jnp
from jax import lax
from jax.experimental import pallas as pl
from jax.experimental.pallas import tpu as pltpu
from jax.experimental.pallas import tpu_sc as plsc

B, C, N_IN = 4, 64, 8192
K, P = 4, 4
N_OUT = 32768
ROWS = B * C
NW = 32
ROWS_PER_W = ROWS // NW
NKROWS = N_IN * K
L = 16
Q = 2048
NQ = N_IN // Q


def _unpool_body(x_hbm, idx_hbm, planes_hbm, out_hbm,
                 acc, xr, irq, gpl, tables_sp, semA, semB):
    nc = 2
    wid = lax.axis_index("s") * nc + lax.axis_index("c")
    lane = jnp.arange(L, dtype=jnp.int32)
    zero = jnp.zeros((L,), jnp.float32)
    lo_mask = jnp.full((L,), 0xFFFF, jnp.int32)
    hi_mask = jnp.full((L,), -65536, jnp.int32)
    sems = [semA, semB]

    @pl.when(lax.axis_index("s") == 0)
    def _():
        pltpu.sync_copy(planes_hbm, tables_sp)

    plsc.subcore_barrier()

    def compute_quarter(q, buf):
        def inner(t, _):
            n0 = t * L
            xv = xr[pl.ds(q * Q + n0, L)]
            for p in range(P):
                wv = gpl[buf, p, pl.ds(n0, L)]
                smv = wv & lo_mask
                iwv = plsc.bitcast(wv & hi_mask, jnp.float32)
                plsc.addupdate_scatter(acc, [smv], xv * iwv)
            return 0

        lax.fori_loop(0, Q // L, inner, 0)

    def row_body(i, _):
        r = wid * ROWS_PER_W + i
        bb = r // C
        cc = r % C
        pltpu.sync_copy(x_hbm.at[bb, cc], xr)

        copies = [None, None]
        for q in range(NQ):
            buf = q % 2
            pltpu.sync_copy(idx_hbm.at[bb, cc, pl.ds(q * Q, Q)], irq.at[buf])

            def gix_body(t, _):
                n0 = t * L
                kv = irq[buf, pl.ds(n0, L)]
                irq[buf, pl.ds(n0, L)] = (lane + (q * Q + n0)) * K + kv
                return 0

            lax.fori_loop(0, Q // L, gix_body, 0)
            cps = []
            for p in range(P):
                cp = pltpu.make_async_copy(
                    tables_sp.at[p].at[irq.at[buf]], gpl.at[buf, p], sems[buf])
                cp.start()
                cps.append(cp)
            copies[buf] = cps
            if q == 0:
                def zbody(j, _):
                    base = j * (L * 8)
                    for u in range(8):
                        acc[pl.ds(base + u * L, L)] = zero
                    return 0

                lax.fori_loop(0, N_OUT // (L * 8), zbody, 0)
            else:
                for cp in copies[1 - buf]:
                    cp.wait()
                compute_quarter(q - 1, 1 - buf)
        for cp in copies[(NQ - 1) % 2]:
            cp.wait()
        compute_quarter(NQ - 1, (NQ - 1) % 2)
        pltpu.sync_copy(acc, out_hbm.at[bb, cc])
        return 0

    lax.fori_loop(0, ROWS_PER_W, row_body, 0)


@jax.jit
def _unpool(x, idx, planes):
    mesh = plsc.VectorSubcoreMesh(core_axis_name="c", subcore_axis_name="s")
    f = functools.partial(
        pl.kernel,
        mesh=mesh,
        compiler_params=pltpu.CompilerParams(
            needs_layout_passes=False, use_tc_tiling_on_sc=False),
        out_type=jax.ShapeDtypeStruct((B, C, N_OUT), jnp.float32),
        scratch_types=[
            pltpu.VMEM((N_OUT,), jnp.float32),
            pltpu.VMEM((N_IN,), jnp.float32),
            pltpu.VMEM((2, Q), jnp.int32),
            pltpu.VMEM((2, P, Q), jnp.int32),
            pltpu.VMEM_SHARED((P, NKROWS), jnp.int32),
            pltpu.SemaphoreType.DMA,
            pltpu.SemaphoreType.DMA,
        ],
    )(_unpool_body)
    return f(x, idx, planes)


def kernel(x, idx_mask, sample_map, interp_weights):
    idxf = idx_mask.astype(jnp.int32)
    sm = sample_map.reshape(NKROWS, P).astype(jnp.int32)
    iw_bits = lax.bitcast_convert_type(
        interp_weights.reshape(NKROWS, P), jnp.int32)
    iw_rounded = (iw_bits + jnp.int32(0x8000)) & jnp.int32(-65536)
    packed = (sm | iw_rounded).T
    return _unpool(x, idxf, packed)


# speedup vs baseline: 1.3682x; 1.0831x over previous
import functools

import jax
import jax.numpy as jnp
from jax import lax
from jax.experimental import pallas as pl
from jax.experimental.pallas import tpu as pltpu
from jax.experimental.pallas import tpu_sc as plsc

B, C, N_IN = 4, 64, 8192
K, P = 4, 4
N_OUT = 32768
ROWS = B * C
NW = 32
ROWS_PER_W = ROWS // NW
NKROWS = N_IN * K
L = 16
Q = 2048
NQ = N_IN // Q


def _unpool_body(x_hbm, gix_hbm, planes_hbm, out_hbm,
                 acc, xrb, irq, gpl, tables_sp,
                 gsemA, gsemB, wsemA, wsemB, psemA, psemB):
    nc = 2
    wid = lax.axis_index("s") * nc + lax.axis_index("c")
    zero = jnp.zeros((L,), jnp.float32)
    lo_mask = jnp.full((L,), 0xFFFF, jnp.int32)
    hi_mask = jnp.full((L,), -65536, jnp.int32)
    gsems = [gsemA, gsemB]
    wsems = [wsemA, wsemB]
    psems = [psemA, psemB]

    @pl.when(lax.axis_index("s") == 0)
    def _():
        pltpu.sync_copy(planes_hbm, tables_sp)

    plsc.subcore_barrier()

    def coords(i):
        r = wid * ROWS_PER_W + i
        return r // C, r % C

    def prefetch(i):
        par = i % 2
        bb, cc = coords(i)
        cps = [pltpu.make_async_copy(x_hbm.at[bb, cc], xrb.at[par],
                                     psems[par])]
        for q in range(NQ):
            cps.append(pltpu.make_async_copy(
                gix_hbm.at[bb, cc, pl.ds(q * Q, Q)], irq.at[par, q],
                psems[par]))
        for cp in cps:
            cp.start()
        return cps

    def compute_quarter(q, buf, par):
        def inner(t, _):
            n0 = t * L
            xv = xrb[par, pl.ds(q * Q + n0, L)]
            for p in range(P):
                wv = gpl[buf, p, pl.ds(n0, L)]
                smv = wv & lo_mask
                iwv = plsc.bitcast(wv & hi_mask, jnp.float32)
                plsc.addupdate_scatter(acc.at[par], [smv], xv * iwv)
            return 0

        lax.fori_loop(0, Q // L, inner, 0)

    pf = prefetch(0)
    wcps = [None, None]
    for i in range(ROWS_PER_W):
        par = i % 2
        bb, cc = coords(i)
        for cp in pf:
            cp.wait()
        gcps = [None, None]
        for q in range(NQ):
            buf = q % 2
            cps = []
            for p in range(P):
                cp = pltpu.make_async_copy(
                    tables_sp.at[p].at[irq.at[par, q]], gpl.at[buf, p],
                    gsems[buf])
                cp.start()
                cps.append(cp)
            gcps[buf] = cps
            if q == 0:
                if wcps[par] is not None:
                    wcps[par].wait()
                    wcps[par] = None

                def zbody(j, _):
                    base = j * (L * 8)
                    for u in range(8):
                        acc[par, pl.ds(base + u * L, L)] = zero
                    return 0

                lax.fori_loop(0, N_OUT // (L * 8), zbody, 0)
            else:
                for cp in gcps[1 - buf]:
                    cp.wait()
                compute_quarter(q - 1, 1 - buf, par)
            if q == 1 and i + 1 < ROWS_PER_W:
                pf = prefetch(i + 1)
        for cp in gcps[(NQ - 1) % 2]:
            cp.wait()
        compute_quarter(NQ - 1, (NQ - 1) % 2, par)
        wcp = pltpu.make_async_copy(acc.at[par], out_hbm.at[bb, cc],
                                    wsems[par])
        wcp.start()
        wcps[par] = wcp
    for wcp in wcps:
        if wcp is not None:
            wcp.wait()


@jax.jit
def _unpool(x, gix, planes):
    mesh = plsc.VectorSubcoreMesh(core_axis_name="c", subcore_axis_name="s")
    f = functools.partial(
        pl.kernel,
        mesh=mesh,
        compiler_params=pltpu.CompilerParams(
            needs_layout_passes=False, use_tc_tiling_on_sc=False),
        out_type=jax.ShapeDtypeStruct((B, C, N_OUT), jnp.float32),
        scratch_types=[
            pltpu.VMEM((2, N_OUT), jnp.float32),
            pltpu.VMEM((2, N_IN), jnp.float32),
            pltpu.VMEM((2, NQ, Q), jnp.int32),
            pltpu.VMEM((2, P, Q), jnp.int32),
            pltpu.VMEM_SHARED((P, NKROWS), jnp.int32),
            pltpu.SemaphoreType.DMA,
            pltpu.SemaphoreType.DMA,
            pltpu.SemaphoreType.DMA,
            pltpu.SemaphoreType.DMA,
            pltpu.SemaphoreType.DMA,
            pltpu.SemaphoreType.DMA,
        ],
    )(_unpool_body)
    return f(x, gix, planes)


def kernel(x, idx_mask, sample_map, interp_weights):
    gix = idx_mask.astype(jnp.int32) + (
        jnp.arange(N_IN, dtype=jnp.int32) * K)[None, None, :]
    sm = sample_map.reshape(NKROWS, P).astype(jnp.int32)
    iw_bits = lax.bitcast_convert_type(
        interp_weights.reshape(NKROWS, P), jnp.int32)
    iw_rounded = (iw_bits + jnp.int32(0x8000)) & jnp.int32(-65536)
    packed = (sm | iw_rounded).T
    return _unpool(x, gix, packed)
